# Initial kernel scaffold; baseline (speedup 1.0000x reference)
import jax
import jax.numpy as jnp
from jax.experimental import pallas as pl


def kernel(node_features, Wh, bh, Wpe, bpe, WQ, bQ, WK, bK, WV, bV, WO, bO, g1, be1, W1, bl1, W2, bl2, g2, be2, mW0, mb0, mW1, mb1, mW2, mb2):
    raise NotImplementedError("write your pallas kernel here")


# speedup vs baseline: 5.1225x; 5.1225x over previous
import functools

import numpy as np
import jax
import jax.numpy as jnp
from jax.experimental import pallas as pl

N = 19
PE_DIM = 8
HIDDEN = 512
INPUT_DIM = 512
OUTPUT_DIM = 16
HEADS = 8
DH = HIDDEN // HEADS
NLAYERS = 3
BATCH = 128

GB = 8
ROWS = GB * N

_A = np.ones((N, N), dtype=np.float64) - np.eye(N)
_Lap = np.diag(_A.sum(axis=1)) - _A
_vals, _vecs = np.linalg.eigh(_Lap)
_PE = _vecs[:, 1:PE_DIM + 1].astype(np.float32)


def _ln(x, g, b):
    mu = jnp.mean(x, axis=-1, keepdims=True)
    var = jnp.mean((x - mu) ** 2, axis=-1, keepdims=True)
    return (x - mu) * jax.lax.rsqrt(var + 1e-5) * g + b


def _fwd_kernel(x_ref, pe_ref, Wh_ref, bh_ref, Wpe_ref, bpe_ref,
                WQ_ref, bQ_ref, WK_ref, bK_ref, WV_ref, bV_ref,
                WO_ref, bO_ref, g1_ref, be1_ref, W1_ref, bl1_ref,
                W2_ref, bl2_ref, g2_ref, be2_ref,
                mW0_ref, mb0_ref, mW1_ref, mb1_ref, mW2_ref, mb2_ref,
                out_ref):
    f32 = jnp.float32
    dot = functools.partial(jnp.dot, preferred_element_type=f32)

    r = jax.lax.broadcasted_iota(jnp.int32, (ROWS, ROWS), 0)
    c = jax.lax.broadcasted_iota(jnp.int32, (ROWS, ROWS), 1)
    same = (r // N) == (c // N)
    edge_mask = jnp.where(same & (r != c), 1.0, 0.0).astype(f32)
    graph_blk = jnp.where(same, 1.0, 0.0).astype(f32)
    r2 = jax.lax.broadcasted_iota(jnp.int32, (ROWS, N), 0)
    c2 = jax.lax.broadcasted_iota(jnp.int32, (ROWS, N), 1)
    tile_pe = jnp.where(r2 % N == c2, 1.0, 0.0).astype(f32)
    r3 = jax.lax.broadcasted_iota(jnp.int32, (GB, ROWS), 0)
    c3 = jax.lax.broadcasted_iota(jnp.int32, (GB, ROWS), 1)
    pool = jnp.where(c3 // N == r3, 1.0 / N, 0.0).astype(f32)

    h_pe = dot(pe_ref[...], Wpe_ref[...]) + bpe_ref[...]
    h = dot(x_ref[...], Wh_ref[...]) + bh_ref[...] + dot(tile_pe, h_pe)

    scale = DH ** -0.5
    for l in range(NLAYERS):
        Q = dot(h, WQ_ref[l]) + bQ_ref[l]
        K = dot(h, WK_ref[l]) + bK_ref[l]
        V = dot(h, WV_ref[l]) + bV_ref[l]
        msgs = []
        for hd in range(HEADS):
            sl = slice(hd * DH, (hd + 1) * DH)
            q = Q[:, sl]
            k = K[:, sl]
            v = V[:, sl]
            s = jax.lax.dot_general(q, k, (((1,), (1,)), ((), ())),
                                    preferred_element_type=f32) * scale
            p = jnp.exp(jnp.clip(s, -5.0, 5.0)) * edge_mask
            rowsum = jnp.sum(p, axis=1, keepdims=True)
            z = dot(graph_blk, rowsum)
            msgs.append(dot(p, v) / z)
        msg = jnp.concatenate(msgs, axis=1)
        h = _ln(h + dot(msg, WO_ref[l]) + bO_ref[l], g1_ref[l], be1_ref[l])
        t = dot(jax.nn.relu(dot(h, W1_ref[l]) + bl1_ref[l]), W2_ref[l]) + bl2_ref[l]
        h = _ln(h + t, g2_ref[l], be2_ref[l])

    pooled = dot(pool, h)
    y = jax.nn.relu(dot(pooled, mW0_ref[...]) + mb0_ref[...])
    y = jax.nn.relu(dot(y, mW1_ref[...]) + mb1_ref[...])
    out_ref[...] = dot(y, mW2_ref[...]) + mb2_ref[...]


def _const_spec(arr):
    nd = arr.ndim
    return pl.BlockSpec(arr.shape, lambda i, _nd=nd: (0,) * _nd)


def kernel(node_features, Wh, bh, Wpe, bpe, WQ, bQ, WK, bK, WV, bV, WO, bO,
           g1, be1, W1, bl1, W2, bl2, g2, be2, mW0, mb0, mW1, mb1, mW2, mb2):
    x = jnp.transpose(node_features, (0, 2, 1)).reshape(BATCH * N, INPUT_DIM)
    pe = jnp.asarray(_PE)

    weights = (Wh, bh, Wpe, bpe, WQ, bQ, WK, bK, WV, bV, WO, bO,
               g1, be1, W1, bl1, W2, bl2, g2, be2,
               mW0, mb0, mW1, mb1, mW2, mb2)
    in_specs = [pl.BlockSpec((ROWS, INPUT_DIM), lambda i: (i, 0)),
                _const_spec(pe)] + [_const_spec(w) for w in weights]

    return pl.pallas_call(
        _fwd_kernel,
        grid=(BATCH // GB,),
        in_specs=in_specs,
        out_specs=pl.BlockSpec((GB, OUTPUT_DIM), lambda i: (i, 0)),
        out_shape=jax.ShapeDtypeStruct((BATCH, OUTPUT_DIM), jnp.float32),
    )(x, pe, *weights)
